# Initial kernel scaffold; baseline (speedup 1.0000x reference)
import jax
import jax.numpy as jnp
from jax.experimental import pallas as pl


def kernel(bboxes, classes, priors):
    raise NotImplementedError("write your pallas kernel here")


# speedup vs baseline: 5.4606x; 5.4606x over previous
import functools

import jax
import jax.numpy as jnp
from jax import lax
from jax.experimental import pallas as pl
from jax.experimental.pallas import tpu as pltpu
from jax.experimental.pallas import tpu_sc as plsc

_VAR0 = 0.1
_VAR1 = 0.2
_THRESHOLD = 0.5
_LN2 = 0.6931471805599453
_SQRT2 = 1.4142135623730951
_L = 16


def _log_f32(x):
    xi = lax.bitcast_convert_type(x, jnp.int32)
    e = lax.shift_right_arithmetic(xi, 23) - 127
    m = lax.bitcast_convert_type(
        lax.bitwise_or(lax.bitwise_and(xi, 0x7FFFFF), 0x3F800000), jnp.float32)
    big = m > _SQRT2
    m = jnp.where(big, 0.5 * m, m)
    e = jnp.where(big, e + 1, e)
    s = (m - 1.0) / (m + 1.0)
    s2 = s * s
    p = 1.0 + s2 * (1.0 / 3.0 + s2 * (0.2 + s2 * (1.0 / 7.0 + s2 * (1.0 / 9.0))))
    return e.astype(jnp.float32) * _LN2 + (2.0 * s) * p


@functools.lru_cache(maxsize=None)
def _build_sc_call(n_pad, n_box, nb_pad, nc, ns):
    nw = nc * ns
    per_w = n_pad // nw
    chunks = per_w // _L
    f32 = jnp.float32
    i32 = jnp.int32

    def body(px1_h, py1_h, px2_h, py2_h, bx1_h, by1_h, bx2_h, by2_h, cls_h,
             ox_h, oy_h, ow_h, oh_h, oc_h,
             px1_v, py1_v, px2_v, py2_v,
             bx1_v, by1_v, bx2_v, by2_v, cls_v, ab_v,
             ox_v, oy_v, ow_v, oh_v, oc_v):
        wid = lax.axis_index("s") * nc + lax.axis_index("c")
        base = pl.multiple_of(wid * per_w, _L)
        pltpu.sync_copy(px1_h.at[pl.ds(base, per_w)], px1_v)
        pltpu.sync_copy(py1_h.at[pl.ds(base, per_w)], py1_v)
        pltpu.sync_copy(px2_h.at[pl.ds(base, per_w)], px2_v)
        pltpu.sync_copy(py2_h.at[pl.ds(base, per_w)], py2_v)
        pltpu.sync_copy(bx1_h, bx1_v)
        pltpu.sync_copy(by1_h, by1_v)
        pltpu.sync_copy(bx2_h, bx2_v)
        pltpu.sync_copy(by2_h, by2_v)
        pltpu.sync_copy(cls_h, cls_v)

        def area_body(k, _):
            o = pl.multiple_of(k * _L, _L)
            ab_v[pl.ds(o, _L)] = (
                (bx2_v[pl.ds(o, _L)] - bx1_v[pl.ds(o, _L)]) *
                (by2_v[pl.ds(o, _L)] - by1_v[pl.ds(o, _L)]))
            return 0

        lax.fori_loop(0, nb_pad // _L, area_body, 0)

        def chunk_body(c, _):
            off = pl.multiple_of(c * _L, _L)
            p1 = px1_v[pl.ds(off, _L)]
            q1 = py1_v[pl.ds(off, _L)]
            p2 = px2_v[pl.ds(off, _L)]
            q2 = py2_v[pl.ds(off, _L)]
            psx = p2 - p1
            psy = q2 - q1
            area_p = psx * psy

            def box_body(i, carry):
                best, bidx = carry
                iv = jnp.full((_L,), i, i32)
                a1 = jnp.maximum(plsc.load_gather(bx1_v, [iv]), p1)
                b1 = jnp.maximum(plsc.load_gather(by1_v, [iv]), q1)
                a2 = jnp.minimum(plsc.load_gather(bx2_v, [iv]), p2)
                b2 = jnp.minimum(plsc.load_gather(by2_v, [iv]), q2)
                iw = jnp.maximum(a2 - a1, 0.0)
                ih = jnp.maximum(b2 - b1, 0.0)
                inter = iw * ih
                iou = inter / ((plsc.load_gather(ab_v, [iv]) + area_p) - inter)
                upd = iou > best
                best = jnp.where(upd, iou, best)
                bidx = jnp.where(upd, i, bidx)
                return best, bidx

            best, bidx = lax.fori_loop(
                0, n_box, box_body,
                (jnp.full((_L,), -1.0, f32), jnp.zeros((_L,), i32)),
                unroll=4)

            gx1 = plsc.load_gather(bx1_v, [bidx])
            gy1 = plsc.load_gather(by1_v, [bidx])
            gx2 = plsc.load_gather(bx2_v, [bidx])
            gy2 = plsc.load_gather(by2_v, [bidx])
            gc = plsc.load_gather(cls_v, [bidx])

            cx = (0.5 * (gx1 + gx2) - 0.5 * (p1 + p2)) / (_VAR0 * psx)
            cy = (0.5 * (gy1 + gy2) - 0.5 * (q1 + q2)) / (_VAR0 * psy)
            w = _log_f32((gx2 - gx1) / psx + 1e-06) / _VAR1
            h = _log_f32((gy2 - gy1) / psy + 1e-06) / _VAR1
            conf = jnp.where(best < _THRESHOLD, 0, 1 + gc)
            ox_v[pl.ds(off, _L)] = cx
            oy_v[pl.ds(off, _L)] = cy
            ow_v[pl.ds(off, _L)] = w
            oh_v[pl.ds(off, _L)] = h
            oc_v[pl.ds(off, _L)] = conf
            return 0

        lax.fori_loop(0, chunks, chunk_body, 0)
        pltpu.sync_copy(ox_v, ox_h.at[pl.ds(base, per_w)])
        pltpu.sync_copy(oy_v, oy_h.at[pl.ds(base, per_w)])
        pltpu.sync_copy(ow_v, ow_h.at[pl.ds(base, per_w)])
        pltpu.sync_copy(oh_v, oh_h.at[pl.ds(base, per_w)])
        pltpu.sync_copy(oc_v, oc_h.at[pl.ds(base, per_w)])

    return pl.kernel(
        body,
        out_type=(
            jax.ShapeDtypeStruct((n_pad,), f32),
            jax.ShapeDtypeStruct((n_pad,), f32),
            jax.ShapeDtypeStruct((n_pad,), f32),
            jax.ShapeDtypeStruct((n_pad,), f32),
            jax.ShapeDtypeStruct((n_pad,), i32),
        ),
        mesh=plsc.VectorSubcoreMesh(core_axis_name="c", subcore_axis_name="s"),
        compiler_params=pltpu.CompilerParams(needs_layout_passes=False),
        scratch_types=[
            pltpu.VMEM((per_w,), f32),
            pltpu.VMEM((per_w,), f32),
            pltpu.VMEM((per_w,), f32),
            pltpu.VMEM((per_w,), f32),
            pltpu.VMEM((nb_pad,), f32),
            pltpu.VMEM((nb_pad,), f32),
            pltpu.VMEM((nb_pad,), f32),
            pltpu.VMEM((nb_pad,), f32),
            pltpu.VMEM((nb_pad,), i32),
            pltpu.VMEM((nb_pad,), f32),
            pltpu.VMEM((per_w,), f32),
            pltpu.VMEM((per_w,), f32),
            pltpu.VMEM((per_w,), f32),
            pltpu.VMEM((per_w,), f32),
            pltpu.VMEM((per_w,), i32),
        ],
    )


def kernel(bboxes, classes, priors):
    n_pri = priors.shape[0]
    n_box = bboxes.shape[0]
    info = plsc.get_sparse_core_info()
    nc, ns = info.num_cores, info.num_subcores
    nw = nc * ns
    grain = nw * _L
    n_pad = ((n_pri + grain - 1) // grain) * grain
    nb_pad = ((n_box + 31) // 32) * 32

    pad_rows = jnp.broadcast_to(
        jnp.array([0.0, 0.0, 1.0, 1.0], jnp.float32), (n_pad - n_pri, 4))
    pri = jnp.concatenate([priors, pad_rows], axis=0)
    bpad = jnp.concatenate(
        [bboxes, jnp.zeros((nb_pad - n_box, 4), jnp.float32)], axis=0)
    cls_pad = jnp.concatenate(
        [classes.astype(jnp.int32), jnp.zeros((nb_pad - n_box,), jnp.int32)])

    fn = _build_sc_call(n_pad, n_box, nb_pad, nc, ns)
    ox, oy, ow, oh, oc = fn(
        pri[:, 0], pri[:, 1], pri[:, 2], pri[:, 3],
        bpad[:, 0], bpad[:, 1], bpad[:, 2], bpad[:, 3],
        cls_pad)
    loc = jnp.stack([ox[:n_pri], oy[:n_pri], ow[:n_pri], oh[:n_pri]], axis=1)
    return loc, oc[:n_pri]
